# Initial kernel scaffold; baseline (speedup 1.0000x reference)
import jax
import jax.numpy as jnp
from jax.experimental import pallas as pl


def kernel(flat_inputs, node_ids, feat_ids):
    raise NotImplementedError("write your pallas kernel here")


# speedup vs baseline: 4.5024x; 4.5024x over previous
import jax
import jax.numpy as jnp
from jax import lax
from jax.experimental import pallas as pl
from jax.experimental.pallas import tpu as pltpu
from jax.experimental.pallas import tpu_sc as plsc

BSZ = 256
D = 8192
N_NODES = 2048
N_FEATS = 64
TOTAL = BSZ * D
CHUNK_WORDS = 65536
N_TASKS = (BSZ * N_NODES * N_FEATS) // CHUNK_WORDS
TASKS_PER_WORKER = N_TASKS // 32
VEC = 16
SEG = 5120
SEG_ITERS = SEG // VEC


def _scatter_body(keys_hbm, vals_hbm, starts_hbm, out_hbm,
                  buf, key_v, val_v, starts_v):
    c = lax.axis_index("c")
    s_ax = lax.axis_index("s")
    wid = s_ax * 2 + c

    lanes = lax.iota(jnp.int32, VEC)
    zeros_f = jnp.zeros((VEC,), jnp.float32)

    pltpu.sync_copy(starts_hbm, starts_v)

    def zero_body(i, carry):
        buf[pl.ds(i * VEC, VEC)] = zeros_f
        return carry

    lax.fori_loop(0, CHUNK_WORDS // VEC, zero_body, 0)

    def task_body(kk, carry):
        t = wid * TASKS_PER_WORKER + kk
        bv = starts_v[pl.ds(pl.multiple_of(t * VEC, VEC), VEC)]
        s0 = jnp.sum(jnp.where(lanes == 0, bv, 0))
        sa = pl.multiple_of(
            jnp.minimum(lax.bitwise_and(s0, -16), TOTAL - SEG), VEC)
        pltpu.sync_copy(keys_hbm.at[pl.ds(sa, SEG)], key_v)
        pltpu.sync_copy(vals_hbm.at[pl.ds(sa, SEG)], val_v)

        def scat_body(i, carry2):
            k = key_v[pl.ds(i * VEC, VEC)]
            v = val_v[pl.ds(i * VEC, VEC)]
            m = lax.shift_right_logical(k, 16) == t
            loc = lax.bitwise_and(k, 0xFFFF)
            plsc.store_scatter(buf, [loc], v, mask=m)
            return carry2

        lax.fori_loop(0, SEG_ITERS, scat_body, 0)

        pltpu.sync_copy(buf, out_hbm.at[t])

        def restore_body(i, carry2):
            k = key_v[pl.ds(i * VEC, VEC)]
            m = lax.shift_right_logical(k, 16) == t
            loc = lax.bitwise_and(k, 0xFFFF)
            plsc.store_scatter(buf, [loc], zeros_f, mask=m)
            return carry2

        lax.fori_loop(0, SEG_ITERS, restore_body, 0)
        return carry

    lax.fori_loop(0, TASKS_PER_WORKER, task_body, 0)


def kernel(flat_inputs, node_ids, feat_ids):
    node = node_ids.astype(jnp.int32)
    feat = feat_ids.astype(jnp.int32)
    slot = (jnp.arange(BSZ, dtype=jnp.int32)[:, None] * (N_NODES * N_FEATS)
            + feat * N_NODES + node).reshape(-1)
    vals = flat_inputs.reshape(-1)
    keys_s, vals_s = lax.sort((slot, vals), num_keys=1, is_stable=False)
    nxt = jnp.concatenate([keys_s[1:], jnp.full((1,), -1, jnp.int32)])
    keys_eff = jnp.where(keys_s != nxt, keys_s, -1)
    targets = jnp.arange(N_TASKS, dtype=jnp.int32) * CHUNK_WORDS
    starts = jnp.searchsorted(keys_s, targets).astype(jnp.int32)
    starts_exp = jnp.pad(starts[:, None], ((0, 0), (0, VEC - 1))).reshape(-1)

    f = pl.kernel(
        _scatter_body,
        out_type=jax.ShapeDtypeStruct((N_TASKS, CHUNK_WORDS), jnp.float32),
        mesh=plsc.VectorSubcoreMesh(core_axis_name="c", subcore_axis_name="s"),
        compiler_params=pltpu.CompilerParams(needs_layout_passes=False),
        scratch_types=[
            pltpu.VMEM((CHUNK_WORDS,), jnp.float32),
            pltpu.VMEM((SEG,), jnp.int32),
            pltpu.VMEM((SEG,), jnp.float32),
            pltpu.VMEM((N_TASKS * VEC,), jnp.int32),
        ],
    )
    out = f(keys_eff, vals_s, starts_exp)
    return out.reshape(BSZ, N_FEATS, N_NODES).transpose(0, 2, 1)


# speedup vs baseline: 4.6376x; 1.0300x over previous
import jax
import jax.numpy as jnp
from jax import lax
from jax.experimental import pallas as pl
from jax.experimental.pallas import tpu as pltpu
from jax.experimental.pallas import tpu_sc as plsc

BSZ = 256
D = 8192
N_NODES = 2048
N_FEATS = 64
TOTAL = BSZ * D
CHUNK_WORDS = 65536
N_TASKS = (BSZ * N_NODES * N_FEATS) // CHUNK_WORDS
TASKS_PER_WORKER = N_TASKS // 32
VEC = 16
SEG = 5120
SEG_ITERS = SEG // VEC


def _scatter_body(keys_hbm, vals_hbm, starts_hbm, out_hbm,
                  buf, k0, v0, k1, v1, starts_v, sem_in, sem_out):
    c = lax.axis_index("c")
    s_ax = lax.axis_index("s")
    wid = s_ax * 2 + c

    lanes = lax.iota(jnp.int32, VEC)
    zeros_f = jnp.zeros((VEC,), jnp.float32)

    pltpu.sync_copy(starts_hbm, starts_v)

    def zero_body(i, carry):
        buf[pl.ds(i * 4 * VEC, VEC)] = zeros_f
        buf[pl.ds(i * 4 * VEC + VEC, VEC)] = zeros_f
        buf[pl.ds(i * 4 * VEC + 2 * VEC, VEC)] = zeros_f
        buf[pl.ds(i * 4 * VEC + 3 * VEC, VEC)] = zeros_f
        return carry

    lax.fori_loop(0, CHUNK_WORDS // (4 * VEC), zero_body, 0)

    def task_scalars(kk):
        t = wid * TASKS_PER_WORKER + kk
        bv = starts_v[pl.ds(pl.multiple_of(t * VEC, VEC), VEC)]
        bv2 = starts_v[pl.ds(pl.multiple_of(t * VEC + VEC, VEC), VEC)]
        s0 = jnp.sum(jnp.where(lanes == 0, bv, 0))
        e0 = jnp.sum(jnp.where(lanes == 0, bv2, 0))
        sa = pl.multiple_of(
            jnp.minimum(lax.bitwise_and(s0, -16), TOTAL - SEG), VEC)
        iters = lax.shift_right_logical(e0 - sa + VEC - 1, 4)
        return t, sa, iters

    bufs = [(k0, v0), (k1, v1)]

    t0, sa0, it0 = task_scalars(0)
    h_k = pltpu.async_copy(keys_hbm.at[pl.ds(sa0, SEG)], k0, sem_in)
    h_v = pltpu.async_copy(vals_hbm.at[pl.ds(sa0, SEG)], v0, sem_in)
    state = (t0, it0, h_k, h_v)

    for kk in range(TASKS_PER_WORKER):
        key_v, val_v = bufs[kk % 2]
        t, iters, h_k, h_v = state
        h_k.wait()
        h_v.wait()

        def scat_body(i, carry2):
            k = key_v[pl.ds(i * VEC, VEC)]
            v = val_v[pl.ds(i * VEC, VEC)]
            m = lax.shift_right_logical(k, 16) == t
            loc = lax.bitwise_and(k, 0xFFFF)
            plsc.store_scatter(buf, [loc], v, mask=m)
            return carry2

        lax.fori_loop(0, iters, scat_body, 0)

        h_out = pltpu.async_copy(buf, out_hbm.at[t], sem_out)

        if kk + 1 < TASKS_PER_WORKER:
            nk, nv = bufs[(kk + 1) % 2]
            t1, sa1, it1 = task_scalars(kk + 1)
            nh_k = pltpu.async_copy(keys_hbm.at[pl.ds(sa1, SEG)], nk, sem_in)
            nh_v = pltpu.async_copy(vals_hbm.at[pl.ds(sa1, SEG)], nv, sem_in)
            state = (t1, it1, nh_k, nh_v)

        h_out.wait()

        def restore_body(i, carry2):
            k = key_v[pl.ds(i * VEC, VEC)]
            m = lax.shift_right_logical(k, 16) == t
            loc = lax.bitwise_and(k, 0xFFFF)
            plsc.store_scatter(buf, [loc], zeros_f, mask=m)
            return carry2

        lax.fori_loop(0, iters, restore_body, 0)


def kernel(flat_inputs, node_ids, feat_ids):
    node = node_ids.astype(jnp.int32)
    feat = feat_ids.astype(jnp.int32)
    slot = (jnp.arange(BSZ, dtype=jnp.int32)[:, None] * (N_NODES * N_FEATS)
            + feat * N_NODES + node).reshape(-1)
    vals = flat_inputs.reshape(-1)
    keys_s, vals_s = lax.sort((slot, vals), num_keys=1, is_stable=False)
    nxt = jnp.concatenate([keys_s[1:], jnp.full((1,), -1, jnp.int32)])
    keys_eff = jnp.where(keys_s != nxt, keys_s, -1)
    targets = jnp.arange(N_TASKS, dtype=jnp.int32) * CHUNK_WORDS
    starts = jnp.searchsorted(keys_s, targets).astype(jnp.int32)
    starts_full = jnp.concatenate(
        [starts, jnp.full((1,), TOTAL, jnp.int32)])
    starts_exp = jnp.pad(
        starts_full[:, None], ((0, 0), (0, VEC - 1))).reshape(-1)

    f = pl.kernel(
        _scatter_body,
        out_type=jax.ShapeDtypeStruct((N_TASKS, CHUNK_WORDS), jnp.float32),
        mesh=plsc.VectorSubcoreMesh(core_axis_name="c", subcore_axis_name="s"),
        compiler_params=pltpu.CompilerParams(needs_layout_passes=False),
        scratch_types=[
            pltpu.VMEM((CHUNK_WORDS,), jnp.float32),
            pltpu.VMEM((SEG,), jnp.int32),
            pltpu.VMEM((SEG,), jnp.float32),
            pltpu.VMEM((SEG,), jnp.int32),
            pltpu.VMEM((SEG,), jnp.float32),
            pltpu.VMEM(((N_TASKS + 1) * VEC,), jnp.int32),
            pltpu.SemaphoreType.DMA,
            pltpu.SemaphoreType.DMA,
        ],
    )
    out = f(keys_eff, vals_s, starts_exp)
    return out.reshape(BSZ, N_FEATS, N_NODES).transpose(0, 2, 1)


# speedup vs baseline: 4.6495x; 1.0026x over previous
import jax
import jax.numpy as jnp
from jax import lax
from jax.experimental import pallas as pl
from jax.experimental.pallas import tpu as pltpu
from jax.experimental.pallas import tpu_sc as plsc

BSZ = 256
D = 8192
N_NODES = 2048
N_FEATS = 64
TOTAL = BSZ * D
CHUNK_WORDS = 65536
N_TASKS = (BSZ * N_NODES * N_FEATS) // CHUNK_WORDS
TASKS_PER_WORKER = N_TASKS // 32
VEC = 16
SEG = 5120
SEG_ITERS = SEG // VEC


def _scatter_body(keys_hbm, vals_hbm, starts_hbm, out_hbm,
                  buf, k0, v0, k1, v1, starts_v, sem_in, sem_out):
    c = lax.axis_index("c")
    s_ax = lax.axis_index("s")
    wid = s_ax * 2 + c

    lanes = lax.iota(jnp.int32, VEC)
    zeros_f = jnp.zeros((VEC,), jnp.float32)

    pltpu.sync_copy(starts_hbm, starts_v)

    def zero_body(i, carry):
        buf[pl.ds(i * 4 * VEC, VEC)] = zeros_f
        buf[pl.ds(i * 4 * VEC + VEC, VEC)] = zeros_f
        buf[pl.ds(i * 4 * VEC + 2 * VEC, VEC)] = zeros_f
        buf[pl.ds(i * 4 * VEC + 3 * VEC, VEC)] = zeros_f
        return carry

    lax.fori_loop(0, CHUNK_WORDS // (4 * VEC), zero_body, 0)

    def task_scalars(kk):
        t = wid * TASKS_PER_WORKER + kk
        bv = starts_v[pl.ds(pl.multiple_of(t * VEC, VEC), VEC)]
        bv2 = starts_v[pl.ds(pl.multiple_of(t * VEC + VEC, VEC), VEC)]
        s0 = jnp.sum(jnp.where(lanes == 0, bv, 0))
        e0 = jnp.sum(jnp.where(lanes == 0, bv2, 0))
        sa = pl.multiple_of(
            jnp.minimum(lax.bitwise_and(s0, -16), TOTAL - SEG), VEC)
        iters = lax.shift_right_logical(e0 - sa + VEC - 1, 4)
        return t, sa, iters

    bufs = [(k0, v0), (k1, v1)]

    t0, sa0, it0 = task_scalars(0)
    h_k = pltpu.async_copy(keys_hbm.at[pl.ds(sa0, SEG)], k0, sem_in)
    h_v = pltpu.async_copy(vals_hbm.at[pl.ds(sa0, SEG)], v0, sem_in)
    state = (t0, it0, h_k, h_v)

    for kk in range(TASKS_PER_WORKER):
        key_v, val_v = bufs[kk % 2]
        t, iters, h_k, h_v = state
        h_k.wait()
        h_v.wait()

        def scat_body(i, carry2):
            k = key_v[pl.ds(i * VEC, VEC)]
            v = val_v[pl.ds(i * VEC, VEC)]
            m = lax.shift_right_logical(k, 16) == t
            loc = lax.bitwise_and(k, 0xFFFF)
            plsc.store_scatter(buf, [loc], v, mask=m)
            return carry2

        lax.fori_loop(0, iters, scat_body, 0)

        h_out = pltpu.async_copy(buf, out_hbm.at[t], sem_out)

        if kk + 1 < TASKS_PER_WORKER:
            nk, nv = bufs[(kk + 1) % 2]
            t1, sa1, it1 = task_scalars(kk + 1)
            nh_k = pltpu.async_copy(keys_hbm.at[pl.ds(sa1, SEG)], nk, sem_in)
            nh_v = pltpu.async_copy(vals_hbm.at[pl.ds(sa1, SEG)], nv, sem_in)
            state = (t1, it1, nh_k, nh_v)

        h_out.wait()

        def restore_body(i, carry2):
            k = key_v[pl.ds(i * VEC, VEC)]
            m = lax.shift_right_logical(k, 16) == t
            loc = lax.bitwise_and(k, 0xFFFF)
            plsc.store_scatter(buf, [loc], zeros_f, mask=m)
            return carry2

        lax.fori_loop(0, iters, restore_body, 0)


def kernel(flat_inputs, node_ids, feat_ids):
    node = node_ids.astype(jnp.int32)
    feat = feat_ids.astype(jnp.int32)
    slot = (jnp.arange(BSZ, dtype=jnp.int32)[:, None] * (N_NODES * N_FEATS)
            + feat * N_NODES + node).reshape(-1)
    vals = flat_inputs.reshape(-1)
    keys_s, vals_s = lax.sort((slot, vals), num_keys=1, is_stable=False)
    nxt = jnp.concatenate([keys_s[1:], jnp.full((1,), -1, jnp.int32)])
    keys_eff = jnp.where(keys_s != nxt, keys_s, -1)
    targets = jnp.arange(N_TASKS, dtype=jnp.int32) * CHUNK_WORDS
    starts = jnp.searchsorted(
        keys_s, targets, method="scan_unrolled").astype(jnp.int32)
    starts_full = jnp.concatenate(
        [starts, jnp.full((1,), TOTAL, jnp.int32)])
    starts_exp = jnp.pad(
        starts_full[:, None], ((0, 0), (0, VEC - 1))).reshape(-1)

    f = pl.kernel(
        _scatter_body,
        out_type=jax.ShapeDtypeStruct((N_TASKS, CHUNK_WORDS), jnp.float32),
        mesh=plsc.VectorSubcoreMesh(core_axis_name="c", subcore_axis_name="s"),
        compiler_params=pltpu.CompilerParams(needs_layout_passes=False),
        scratch_types=[
            pltpu.VMEM((CHUNK_WORDS,), jnp.float32),
            pltpu.VMEM((SEG,), jnp.int32),
            pltpu.VMEM((SEG,), jnp.float32),
            pltpu.VMEM((SEG,), jnp.int32),
            pltpu.VMEM((SEG,), jnp.float32),
            pltpu.VMEM(((N_TASKS + 1) * VEC,), jnp.int32),
            pltpu.SemaphoreType.DMA,
            pltpu.SemaphoreType.DMA,
        ],
    )
    out = f(keys_eff, vals_s, starts_exp)
    return out.reshape(BSZ, N_FEATS, N_NODES).transpose(0, 2, 1)


# speedup vs baseline: 4.6523x; 1.0006x over previous
import jax
import jax.numpy as jnp
from jax import lax
from jax.experimental import pallas as pl
from jax.experimental.pallas import tpu as pltpu
from jax.experimental.pallas import tpu_sc as plsc

BSZ = 256
D = 8192
N_NODES = 2048
N_FEATS = 64
TOTAL = BSZ * D
CHUNK_WORDS = 65536
N_TASKS = (BSZ * N_NODES * N_FEATS) // CHUNK_WORDS
TASKS_PER_WORKER = N_TASKS // 32
VEC = 16
SEG = 5120
SEG_ITERS = SEG // VEC


def _scatter_body(keys_hbm, vals_hbm, starts_hbm, out_hbm,
                  buf, k0, v0, k1, v1, starts_v, sem_in, sem_out):
    c = lax.axis_index("c")
    s_ax = lax.axis_index("s")
    wid = s_ax * 2 + c

    lanes = lax.iota(jnp.int32, VEC)
    zeros_f = jnp.zeros((VEC,), jnp.float32)

    pltpu.sync_copy(starts_hbm, starts_v)

    def zero_body(i, carry):
        buf[pl.ds(i * 4 * VEC, VEC)] = zeros_f
        buf[pl.ds(i * 4 * VEC + VEC, VEC)] = zeros_f
        buf[pl.ds(i * 4 * VEC + 2 * VEC, VEC)] = zeros_f
        buf[pl.ds(i * 4 * VEC + 3 * VEC, VEC)] = zeros_f
        return carry

    lax.fori_loop(0, CHUNK_WORDS // (4 * VEC), zero_body, 0)

    def task_scalars(kk):
        t = wid * TASKS_PER_WORKER + kk
        bv = starts_v[pl.ds(pl.multiple_of(t * VEC, VEC), VEC)]
        bv2 = starts_v[pl.ds(pl.multiple_of(t * VEC + VEC, VEC), VEC)]
        s0 = jnp.sum(jnp.where(lanes == 0, bv, 0))
        e0 = jnp.sum(jnp.where(lanes == 0, bv2, 0))
        sa = pl.multiple_of(
            jnp.minimum(lax.bitwise_and(s0, -16), TOTAL - SEG), VEC)
        iters = lax.shift_right_logical(e0 - sa + VEC - 1, 4)
        return t, sa, iters

    bufs = [(k0, v0), (k1, v1)]

    t0, sa0, it0 = task_scalars(0)
    h_k = pltpu.async_copy(keys_hbm.at[pl.ds(sa0, SEG)], k0, sem_in)
    h_v = pltpu.async_copy(vals_hbm.at[pl.ds(sa0, SEG)], v0, sem_in)
    state = (t0, it0, h_k, h_v)

    for kk in range(TASKS_PER_WORKER):
        key_v, val_v = bufs[kk % 2]
        t, iters, h_k, h_v = state
        h_k.wait()
        h_v.wait()

        def scat_body(i, carry2):
            k = key_v[pl.ds(i * VEC, VEC)]
            v = val_v[pl.ds(i * VEC, VEC)]
            m = lax.shift_right_logical(k, 16) == t
            loc = lax.bitwise_and(k, 0xFFFF)
            plsc.store_scatter(buf, [loc], v, mask=m)
            return carry2

        lax.fori_loop(0, iters, scat_body, 0)

        h_out = pltpu.async_copy(buf, out_hbm.at[t], sem_out)

        if kk + 1 < TASKS_PER_WORKER:
            nk, nv = bufs[(kk + 1) % 2]
            t1, sa1, it1 = task_scalars(kk + 1)
            nh_k = pltpu.async_copy(keys_hbm.at[pl.ds(sa1, SEG)], nk, sem_in)
            nh_v = pltpu.async_copy(vals_hbm.at[pl.ds(sa1, SEG)], nv, sem_in)
            state = (t1, it1, nh_k, nh_v)

        h_out.wait()

        if kk + 1 < TASKS_PER_WORKER:
            def restore_body(i, carry2):
                k = key_v[pl.ds(i * VEC, VEC)]
                m = lax.shift_right_logical(k, 16) == t
                loc = lax.bitwise_and(k, 0xFFFF)
                plsc.store_scatter(buf, [loc], zeros_f, mask=m)
                return carry2

            lax.fori_loop(0, iters, restore_body, 0)


def kernel(flat_inputs, node_ids, feat_ids):
    node = node_ids.astype(jnp.int32)
    feat = feat_ids.astype(jnp.int32)
    slot = (jnp.arange(BSZ, dtype=jnp.int32)[:, None] * (N_NODES * N_FEATS)
            + feat * N_NODES + node).reshape(-1)
    vals = flat_inputs.reshape(-1)
    keys_s, vals_s = lax.sort((slot, vals), num_keys=1, is_stable=False)
    nxt = jnp.concatenate([keys_s[1:], jnp.full((1,), -1, jnp.int32)])
    keys_eff = jnp.where(keys_s != nxt, keys_s, -1)
    targets = jnp.arange(N_TASKS, dtype=jnp.int32) * CHUNK_WORDS
    starts = jnp.searchsorted(
        keys_s, targets, method="scan_unrolled").astype(jnp.int32)
    starts_full = jnp.concatenate(
        [starts, jnp.full((1,), TOTAL, jnp.int32)])
    starts_exp = jnp.pad(
        starts_full[:, None], ((0, 0), (0, VEC - 1))).reshape(-1)

    f = pl.kernel(
        _scatter_body,
        out_type=jax.ShapeDtypeStruct((N_TASKS, CHUNK_WORDS), jnp.float32),
        mesh=plsc.VectorSubcoreMesh(core_axis_name="c", subcore_axis_name="s"),
        compiler_params=pltpu.CompilerParams(needs_layout_passes=False),
        scratch_types=[
            pltpu.VMEM((CHUNK_WORDS,), jnp.float32),
            pltpu.VMEM((SEG,), jnp.int32),
            pltpu.VMEM((SEG,), jnp.float32),
            pltpu.VMEM((SEG,), jnp.int32),
            pltpu.VMEM((SEG,), jnp.float32),
            pltpu.VMEM(((N_TASKS + 1) * VEC,), jnp.int32),
            pltpu.SemaphoreType.DMA,
            pltpu.SemaphoreType.DMA,
        ],
    )
    out = f(keys_eff, vals_s, starts_exp)
    return out.reshape(BSZ, N_FEATS, N_NODES).transpose(0, 2, 1)
